# baseline (speedup 1.0000x reference)
import functools

import jax
import jax.numpy as jnp
from jax import lax
from jax.experimental import pallas as pl
from jax.experimental.pallas import tpu as pltpu
from jax.experimental.pallas import tpu_sc as plsc

NC = 2
NS = 16
LANES = 16


@functools.lru_cache(maxsize=None)
def _build(num_nodes, feat, num_edges):
    et = -(-num_edges // (NS * LANES)) * LANES
    pad_edges = et * NS
    hch = -(-(num_nodes + 1) // (NS * LANES)) * LANES
    hist = hch * NS
    tail = num_nodes - (NS - 1) * hch

    mesh = plsc.VectorSubcoreMesh(
        core_axis_name="c", subcore_axis_name="s", num_cores=NC, num_subcores=NS
    )

    @functools.partial(
        pl.kernel,
        out_type=jax.ShapeDtypeStruct((num_nodes,), jnp.float32),
        mesh=mesh,
        scratch_types=[
            pltpu.VMEM((et,), jnp.int32),
            pltpu.VMEM((et,), jnp.float32),
            pltpu.VMEM((hch,), jnp.float32),
            pltpu.VMEM((LANES,), jnp.float32),
            pltpu.VMEM((NS * LANES,), jnp.float32),
            pltpu.VMEM_SHARED((hist,), jnp.float32),
            pltpu.VMEM_SHARED((NS * LANES,), jnp.float32),
            pltpu.SemaphoreType.DMA,
            pltpu.SemaphoreType.DMA,
        ],
    )
    def sc_deg(rows_hbm, ones_hbm, deg_hbm, idx_v, ones_v, ch_v, mv, am, hist_s,
               maxs_s, sem1, sem2):
        c = lax.axis_index("c")
        s = lax.axis_index("s")
        zero16 = jnp.zeros((LANES,), jnp.float32)

        cp1 = pltpu.async_copy(rows_hbm.at[pl.ds(s * et, et)], idx_v, sem1)
        cp2 = pltpu.async_copy(ones_hbm, ones_v, sem2)

        def fill_zeros(i, carry):
            for k in range(5):
                ch_v[pl.ds((i * 5 + k) * LANES, LANES)] = zero16
            return carry

        lax.fori_loop(0, hch // (5 * LANES), fill_zeros, 0)
        pltpu.sync_copy(ch_v, hist_s.at[pl.ds(s * hch, hch)])
        cp1.wait()
        cp2.wait()
        plsc.subcore_barrier()
        pltpu.sync_copy(ones_v, hist_s.at[idx_v], add=True)
        plsc.subcore_barrier()

        pltpu.sync_copy(hist_s.at[pl.ds(s * hch, hch)], ch_v)
        lanes = lax.broadcasted_iota(jnp.int32, (LANES,), 0)

        def chunk_max(i, mx):
            v = ch_v[pl.ds(i * LANES, LANES)]
            gidx = s * hch + i * LANES + lanes
            return jnp.maximum(mx, jnp.where(gidx < num_nodes, v, 0.0))

        mx = lax.fori_loop(0, hch // LANES, chunk_max, jnp.zeros((LANES,), jnp.float32))
        mv[pl.ds(0, LANES)] = mx
        pltpu.sync_copy(mv, maxs_s.at[pl.ds(s * LANES, LANES)])
        plsc.subcore_barrier()
        pltpu.sync_copy(maxs_s, am)

        def tile_max(j, m2):
            return jnp.maximum(m2, am[pl.ds(j * LANES, LANES)])

        mx2 = lax.fori_loop(0, NS, tile_max, jnp.zeros((LANES,), jnp.float32))
        dnums = lax.GatherDimensionNumbers(
            offset_dims=(), collapsed_slice_dims=(0,), start_index_map=(0,)
        )
        for shift in (1, 2, 4, 8):
            shuf = lax.gather(
                mx2,
                (lanes ^ shift)[:, None],
                dnums,
                slice_sizes=(1,),
                mode=lax.GatherScatterMode.PROMISE_IN_BOUNDS,
            )
            mx2 = jnp.maximum(mx2, shuf)
        inv = 1.0 / mx2

        def scale(i, carry):
            ch_v[pl.ds(i * LANES, LANES)] = ch_v[pl.ds(i * LANES, LANES)] * inv
            return carry

        lax.fori_loop(0, hch // LANES, scale, 0)

        @pl.when((c == 0) & (s < NS - 1))
        def _():
            pltpu.sync_copy(ch_v, deg_hbm.at[pl.ds(s * hch, hch)])

        @pl.when((c == 0) & (s == NS - 1))
        def _():
            pltpu.sync_copy(
                ch_v.at[pl.ds(0, tail)], deg_hbm.at[pl.ds((NS - 1) * hch, tail)]
            )

    rb = 1000

    def cat_body(x_ref, d_ref, o_ref):
        o_ref[:, :feat] = x_ref[...]
        o_ref[:, feat : feat + 1] = d_ref[...]

    tc_concat = pl.pallas_call(
        cat_body,
        grid=(num_nodes // rb,),
        in_specs=[
            pl.BlockSpec((rb, feat), lambda i: (i, 0)),
            pl.BlockSpec((rb, 1), lambda i: (i, 0)),
        ],
        out_specs=pl.BlockSpec((rb, feat + 1), lambda i: (i, 0)),
        out_shape=jax.ShapeDtypeStruct((num_nodes, feat + 1), jnp.float32),
    )

    def run(x, edge_index):
        row = edge_index[0].astype(jnp.int32)
        pad = jnp.full((pad_edges - num_edges,), num_nodes, jnp.int32)
        rows = jnp.concatenate([row, pad])
        ones = jnp.ones((et,), jnp.float32)
        deg = sc_deg(rows, ones)
        return tc_concat(x, deg.reshape(num_nodes, 1))

    return run


def kernel(x, edge_index):
    return _build(x.shape[0], x.shape[1], edge_index.shape[1])(x, edge_index)


# speedup vs baseline: 1.1257x; 1.1257x over previous
import functools

import jax
import jax.numpy as jnp
from jax import lax
from jax.experimental import pallas as pl
from jax.experimental.pallas import tpu as pltpu
from jax.experimental.pallas import tpu_sc as plsc

NC = 2
NS = 16
LANES = 16


@functools.lru_cache(maxsize=None)
def _build(num_nodes, feat, num_edges):
    et = -(-num_edges // (NS * LANES)) * LANES
    pad_edges = et * NS
    direct = pad_edges == num_edges
    hch = -(-(num_nodes + 1) // (NS * LANES)) * LANES
    hist = hch * NS
    tail = num_nodes - (NS - 1) * hch

    mesh = plsc.VectorSubcoreMesh(
        core_axis_name="c", subcore_axis_name="s", num_cores=NC, num_subcores=NS
    )

    @functools.partial(
        pl.kernel,
        out_type=jax.ShapeDtypeStruct((num_nodes,), jnp.float32),
        mesh=mesh,
        scratch_types=[
            pltpu.VMEM((et,), jnp.int32),
            pltpu.VMEM((et,), jnp.float32),
            pltpu.VMEM((hch,), jnp.float32),
            pltpu.VMEM((LANES,), jnp.float32),
            pltpu.VMEM((NS * LANES,), jnp.float32),
            pltpu.VMEM_SHARED((hist,), jnp.float32),
            pltpu.VMEM_SHARED((NS * LANES,), jnp.float32),
            pltpu.SemaphoreType.DMA,
            pltpu.SemaphoreType.DMA,
        ],
    )
    def sc_deg(rows_hbm, ones_hbm, deg_hbm, idx_v, ones_v, ch_v, mv, am, hist_s,
               maxs_s, sem1, sem2):
        c = lax.axis_index("c")
        s = lax.axis_index("s")
        zero16 = jnp.zeros((LANES,), jnp.float32)

        cp1 = pltpu.async_copy(rows_hbm.at[pl.ds(s * et, et)], idx_v, sem1)
        cp2 = pltpu.async_copy(ones_hbm, ones_v, sem2)

        def fill_zeros(i, carry):
            for k in range(5):
                ch_v[pl.ds((i * 5 + k) * LANES, LANES)] = zero16
            return carry

        lax.fori_loop(0, hch // (5 * LANES), fill_zeros, 0)
        pltpu.sync_copy(ch_v, hist_s.at[pl.ds(s * hch, hch)])
        cp1.wait()
        cp2.wait()
        plsc.subcore_barrier()
        pltpu.sync_copy(ones_v, hist_s.at[idx_v], add=True)
        plsc.subcore_barrier()

        pltpu.sync_copy(hist_s.at[pl.ds(s * hch, hch)], ch_v)
        lanes = lax.broadcasted_iota(jnp.int32, (LANES,), 0)

        def chunk_max(i, mx):
            v = ch_v[pl.ds(i * LANES, LANES)]
            gidx = s * hch + i * LANES + lanes
            return jnp.maximum(mx, jnp.where(gidx < num_nodes, v, 0.0))

        mx = lax.fori_loop(0, hch // LANES, chunk_max, jnp.zeros((LANES,), jnp.float32))
        mv[pl.ds(0, LANES)] = mx
        pltpu.sync_copy(mv, maxs_s.at[pl.ds(s * LANES, LANES)])
        plsc.subcore_barrier()
        pltpu.sync_copy(maxs_s, am)

        def tile_max(j, m2):
            return jnp.maximum(m2, am[pl.ds(j * LANES, LANES)])

        mx2 = lax.fori_loop(0, NS, tile_max, jnp.zeros((LANES,), jnp.float32))
        dnums = lax.GatherDimensionNumbers(
            offset_dims=(), collapsed_slice_dims=(0,), start_index_map=(0,)
        )
        for shift in (1, 2, 4, 8):
            shuf = lax.gather(
                mx2,
                (lanes ^ shift)[:, None],
                dnums,
                slice_sizes=(1,),
                mode=lax.GatherScatterMode.PROMISE_IN_BOUNDS,
            )
            mx2 = jnp.maximum(mx2, shuf)
        inv = 1.0 / mx2

        def scale(i, carry):
            ch_v[pl.ds(i * LANES, LANES)] = ch_v[pl.ds(i * LANES, LANES)] * inv
            return carry

        lax.fori_loop(0, hch // LANES, scale, 0)

        @pl.when((c == 0) & (s < NS - 1))
        def _():
            pltpu.sync_copy(ch_v, deg_hbm.at[pl.ds(s * hch, hch)])

        @pl.when((c == 0) & (s == NS - 1))
        def _():
            pltpu.sync_copy(
                ch_v.at[pl.ds(0, tail)], deg_hbm.at[pl.ds((NS - 1) * hch, tail)]
            )

    rb = 1000

    def cat_body(x_ref, d_ref, o_hbm, stage, sem):
        i = pl.program_id(0)
        stage[:, :feat] = x_ref[...]
        stage[:, feat : feat + 1] = d_ref[...]
        pltpu.async_copy(stage, o_hbm.at[pl.ds(i * rb, rb), :], sem).wait()

    tc_concat = pl.pallas_call(
        cat_body,
        grid=(num_nodes // rb,),
        in_specs=[
            pl.BlockSpec((rb, feat), lambda i: (i, 0)),
            pl.BlockSpec((rb, 1), lambda i: (i, 0)),
        ],
        out_specs=pl.BlockSpec(memory_space=pltpu.MemorySpace.HBM),
        out_shape=jax.ShapeDtypeStruct((num_nodes, feat + 1), jnp.float32),
        scratch_shapes=[
            pltpu.VMEM((rb, feat + 1), jnp.float32),
            pltpu.SemaphoreType.DMA,
        ],
    )

    def run(x, edge_index):
        if direct and edge_index.dtype == jnp.int32:
            rows = edge_index.reshape(-1)
        else:
            row = edge_index[0].astype(jnp.int32)
            pad = jnp.full((pad_edges - num_edges,), num_nodes, jnp.int32)
            rows = jnp.concatenate([row, pad])
        ones = jnp.ones((et,), jnp.float32)
        deg = sc_deg(rows, ones)
        return tc_concat(x, deg.reshape(num_nodes, 1))

    return run


def kernel(x, edge_index):
    return _build(x.shape[0], x.shape[1], edge_index.shape[1])(x, edge_index)
